# baseline (speedup 1.0000x reference)
import jax
import jax.numpy as jnp
from jax.experimental import pallas as pl
from jax.experimental.pallas import tpu as pltpu

POSITIONS = 200
CHUNK = 1024
NCHUNK = 16


def _onehot_block(in_ref, out_ref):
    vals = in_ref[0, 0, :]
    cols = jax.lax.broadcasted_iota(jnp.int32, (CHUNK, POSITIONS), 1)
    out_ref[:, :] = (vals[:, None] == cols).astype(jnp.float32)


def kernel(inputs):
    n = inputs.shape[0]
    inputs3 = inputs.reshape(NCHUNK, 1, CHUNK)
    return pl.pallas_call(
        _onehot_block,
        grid=(NCHUNK,),
        in_specs=[pl.BlockSpec((1, 1, CHUNK), lambda i: (i, 0, 0))],
        out_specs=pl.BlockSpec((CHUNK, POSITIONS), lambda i: (i, 0)),
        out_shape=jax.ShapeDtypeStruct((n, POSITIONS), jnp.float32),
    )(inputs3)


# speedup vs baseline: 1.0033x; 1.0033x over previous
import jax
import jax.numpy as jnp
from jax.experimental import pallas as pl
from jax.experimental.pallas import tpu as pltpu

POSITIONS = 200
CHUNK = 1024
NCHUNK = 16


def _onehot_block(in_ref, out_ref):
    vals = in_ref[0, 0, :]
    cols = jax.lax.broadcasted_iota(jnp.int32, (CHUNK, POSITIONS), 1)
    out_ref[:, :] = (vals[:, None] == cols).astype(jnp.float32)


def kernel(inputs):
    n = inputs.shape[0]
    inputs3 = inputs.reshape(NCHUNK, 1, CHUNK)
    return pl.pallas_call(
        _onehot_block,
        grid=(NCHUNK,),
        in_specs=[pl.BlockSpec((1, 1, CHUNK), lambda i: (i, 0, 0))],
        out_specs=pl.BlockSpec((CHUNK, POSITIONS), lambda i: (i, 0)),
        out_shape=jax.ShapeDtypeStruct((n, POSITIONS), jnp.float32),
        compiler_params=pltpu.CompilerParams(
            dimension_semantics=("parallel",),
        ),
    )(inputs3)


# speedup vs baseline: 3.7729x; 3.7606x over previous
import jax
import jax.numpy as jnp
from jax.experimental import pallas as pl
from jax.experimental.pallas import tpu as pltpu

POSITIONS = 200
CHUNK = 2048
NCHUNK = 8


def _onehot_t_block(in_ref, out_ref):
    vals = in_ref[0, 0, :]
    rows = jax.lax.broadcasted_iota(jnp.int32, (POSITIONS, CHUNK), 0)
    out_ref[:, :] = (vals[None, :] == rows).astype(jnp.float32)


def kernel(inputs):
    n = inputs.shape[0]
    inputs3 = inputs.reshape(NCHUNK, 1, CHUNK)
    out_t = pl.pallas_call(
        _onehot_t_block,
        grid=(NCHUNK,),
        in_specs=[pl.BlockSpec((1, 1, CHUNK), lambda i: (i, 0, 0))],
        out_specs=pl.BlockSpec((POSITIONS, CHUNK), lambda i: (0, i)),
        out_shape=jax.ShapeDtypeStruct((POSITIONS, n), jnp.float32),
        compiler_params=pltpu.CompilerParams(
            dimension_semantics=("parallel",),
        ),
    )(inputs3)
    return out_t.T


# speedup vs baseline: 4.5785x; 1.2135x over previous
import jax
import jax.numpy as jnp
from jax.experimental import pallas as pl
from jax.experimental.pallas import tpu as pltpu

POSITIONS = 200
CHUNK = 4096
NCHUNK = 4


def _onehot_t_block(in_ref, out_ref):
    vals = in_ref[0, 0, :]
    rows = jax.lax.broadcasted_iota(jnp.int32, (POSITIONS, CHUNK), 0)
    out_ref[:, :] = (vals[None, :] == rows).astype(jnp.float32)


def kernel(inputs):
    n = inputs.shape[0]
    inputs3 = inputs.reshape(NCHUNK, 1, CHUNK)
    out_t = pl.pallas_call(
        _onehot_t_block,
        grid=(NCHUNK,),
        in_specs=[pl.BlockSpec((1, 1, CHUNK), lambda i: (i, 0, 0))],
        out_specs=pl.BlockSpec((POSITIONS, CHUNK), lambda i: (0, i)),
        out_shape=jax.ShapeDtypeStruct((POSITIONS, n), jnp.float32),
        compiler_params=pltpu.CompilerParams(
            dimension_semantics=("parallel",),
        ),
    )(inputs3)
    return out_t.T
